# baseline (speedup 1.0000x reference)
import functools

import jax
import jax.numpy as jnp
import numpy as np
from jax import lax
from jax.experimental import pallas as pl
from jax.experimental.pallas import tpu as pltpu
from jax.experimental.pallas import tpu_sc as plsc

HIDDEN = 256
NUM_HEAD = 4
BLOCKS = 3
INPUT_DIM = 1024
OUTPUT_DIM = 256
BIN_SIZE_D = 0.2
BIN_SIZE_A = 15.0
ANGLE_K = 3
MAX_NEIGHBOOR = 16
GEODESIC_RADIS = 0.5
BIN_FACTOR_A = 180.0 / (BIN_SIZE_A * np.pi)
BIN_SIZE_GEO = BIN_SIZE_D / 2.0


def _knn_body(pts_ref, ptsT_ref, nkd_ref, nki_ref):
    p = pts_ref[...]
    pt = ptsT_ref[...]
    n = p.shape[0]
    d2 = jnp.zeros((n, n), jnp.float32)
    for d in range(3):
        diff = p[:, d : d + 1] - pt[d : d + 1, :]
        d2 = d2 + diff * diff
    cols = lax.broadcasted_iota(jnp.int32, (n, n), 1)
    dist = jnp.sqrt(jnp.maximum(d2, 1e-12))
    for k in range(MAX_NEIGHBOOR):
        m = jnp.min(dist, axis=1, keepdims=True)
        am = jnp.min(jnp.where(dist <= m, cols, n), axis=1, keepdims=True)
        nkd_ref[:, k : k + 1] = m
        nki_ref[:, k : k + 1] = am
        dist = jnp.where(cols == am, jnp.inf, dist)


def _knn(points, interpret=False):
    n = points.shape[0]
    return pl.pallas_call(
        _knn_body,
        out_shape=(
            jax.ShapeDtypeStruct((n, MAX_NEIGHBOOR), jnp.float32),
            jax.ShapeDtypeStruct((n, MAX_NEIGHBOOR), jnp.int32),
        ),
        interpret=interpret,
    )(points, points.T)


def _sin_embed(x, d=HIDDEN):
    div = jnp.exp(jnp.arange(0, d, 2, dtype=jnp.float32) * (-np.log(10000.0) / d))
    om = x[..., None] * div
    e = jnp.stack([jnp.sin(om), jnp.cos(om)], axis=-1)
    return e.reshape(x.shape + (d,))


_N = 1024
_K = MAX_NEIGHBOOR
_NW = 32
_CPW = _N // _NW


def _geo_sc_body(edgeT_hbm, idxT_hbm, out_hbm, edge_v, idx_v, a_v, b_v, chg_v,
                 sem):
    wid = lax.axis_index("s") * 2 + lax.axis_index("c")
    pltpu.sync_copy(edgeT_hbm, edge_v)
    pltpu.sync_copy(idxT_hbm, idx_v)
    lanes = jax.lax.iota(jnp.int32, 16)
    inf16 = jnp.full((16,), jnp.inf, jnp.float32)

    def init_g(g, _):
        rows = lanes + g * 16
        for c in range(_CPW):
            a_v[pl.ds(c * _N + g * 16, 16)] = jnp.where(
                rows == wid * _CPW + c, 0.0, inf16)
        return 0
    lax.fori_loop(0, _N // 16, init_g, 0)

    def relax(s_ref, d_ref, tracked):
        def g_body(g, chg):
            i0 = g * 16
            nv = [idx_v[k, pl.ds(i0, 16)] for k in range(_K)]
            ev = [edge_v[k, pl.ds(i0, 16)] for k in range(_K)]

            def c_body(cc, chg):
                base = cc * _N
                bvec = jnp.full((16,), base, jnp.int32)
                cand = [
                    ev[k] + plsc.load_gather(s_ref, [nv[k] + bvec])
                    for k in range(_K)
                ]
                while len(cand) > 1:
                    cand = [jnp.minimum(cand[2 * i], cand[2 * i + 1])
                            for i in range(len(cand) // 2)]
                old = s_ref[pl.ds(base + i0, 16)]
                acc = jnp.minimum(old, cand[0])
                d_ref[pl.ds(base + i0, 16)] = acc
                if tracked:
                    chg = jnp.where(acc != old, jnp.int32(1), chg)
                return chg

            return lax.fori_loop(0, _CPW, c_body, chg)

        return lax.fori_loop(0, _N // 16, g_body,
                             jnp.zeros((16,), jnp.int32))

    chg_v[...] = jnp.ones((16,), jnp.int32)

    def pair_body(t, conv):
        @pl.when(conv > 0)
        def _():
            relax(a_v, b_v, False)
            chg_v[...] = relax(b_v, a_v, True)

        return jnp.max(chg_v[...])

    lax.fori_loop(0, 16, pair_body, jnp.int32(1))
    pltpu.sync_copy(a_v, out_hbm.at[pl.ds(wid * _CPW * _N, _CPW * _N)])


def _geodesic_sc(edge_d, nki, interpret=False):
    mesh = plsc.VectorSubcoreMesh(core_axis_name="c", subcore_axis_name="s")
    kern = pl.kernel(
        _geo_sc_body,
        out_type=jax.ShapeDtypeStruct((_N * _N,), jnp.float32),
        mesh=mesh,
        compiler_params=pltpu.CompilerParams(
            needs_layout_passes=False, use_tc_tiling_on_sc=False),
        scratch_types=[
            pltpu.VMEM((_K, _N), jnp.float32),
            pltpu.VMEM((_K, _N), jnp.int32),
            pltpu.VMEM((_CPW * _N,), jnp.float32),
            pltpu.VMEM((_CPW * _N,), jnp.float32),
            pltpu.VMEM((16,), jnp.int32),
            pltpu.SemaphoreType.DMA,
        ],
        interpret=interpret,
    )
    return kern(edge_d.T, nki.T).reshape(_N, _N)


def _geodesic(nkd, nki, max_step=32):
    n = nkd.shape[0]
    edge_d = jnp.where(nkd <= GEODESIC_RADIS, nkd, jnp.inf)
    geoT = _geodesic_sc(edge_d, nki)
    return jnp.where(jnp.isinf(geoT), -1.0, geoT)


def _ln(x, g, b):
    mu = x.mean(-1, keepdims=True)
    v = ((x - mu) ** 2).mean(-1, keepdims=True)
    return (x - mu) / jnp.sqrt(v + 1e-5) * g + b


def _geom_embed(points, nkd, nki, p, geodesic):
    masks = None
    nkp = points[nki]
    rde_idx = nkd / BIN_SIZE_D
    knn_pts = nkp[:, 1 : 1 + ANGLE_K, :]
    ref_vec = knn_pts - points[:, None, :]
    anc_vec = nkp[:, None, :, :] - points[:, None, None, :]
    ref_vec = ref_vec[nki][:, None]
    anc_vec = jnp.broadcast_to(anc_vec[:, :, :, None, :], ref_vec.shape)
    sinv = jnp.linalg.norm(jnp.cross(ref_vec, anc_vec), axis=-1)
    cosv = jnp.sum(ref_vec * anc_vec, axis=-1)
    rae_idx = jnp.arctan2(sinv, cosv) * BIN_FACTOR_A
    rde = _sin_embed(rde_idx[:, None]) @ p['rde_w'] + p['rde_b']
    rae = _sin_embed(rae_idx) @ p['rae_w'] + p['rae_b']
    rae = rae.max(axis=3)
    rge = rde + rae
    if geodesic:
        geoW = _geodesic(nkd, nki)
        mctx = geoW.max(axis=0)
        mval = mctx.max()
        mctx = jnp.where(mctx < 0, mval, mctx)
        gd0 = jnp.take_along_axis(geoW, nki.T, axis=0).T
        gd = jnp.where(gd0 < 0, mctx[:, None], gd0) / BIN_SIZE_GEO
        rdi = _sin_embed(gd[:, None]) @ p['rdi_w'] + p['rdi_b'] + rde
        return rge, rdi, masks
    return rge, masks


def _self_block(x, emb, knn_oh, p, i):
    n, d = x.shape
    h = NUM_HEAD
    dh = d // h
    q = x @ p['sa_wq'][i]
    k = x @ p['sa_wk'][i]
    v = x @ p['sa_wv'][i]
    pe = emb[:, 0] @ p['sa_wp'][i]
    kn = jnp.einsum('nkj,jd->nkd', knn_oh, k,
                    precision=jax.lax.Precision.HIGHEST) + pe
    vn = jnp.einsum('nkj,jd->nkd', knn_oh, v,
                    precision=jax.lax.Precision.HIGHEST)
    qh = q.reshape(n, h, dh)
    knh = kn.reshape(n, -1, h, dh)
    vnh = vn.reshape(n, -1, h, dh)
    att = jax.nn.softmax(jnp.einsum('nhd,nkhd->nhk', qh, knh) / np.sqrt(dh), axis=-1)
    o = jnp.einsum('nhk,nkhd->nhd', att, vnh).reshape(n, d) @ p['sa_wo'][i]
    x = _ln(x + o, p['sa_n1g'][i], p['sa_n1b'][i])
    hdn = jax.nn.relu(x @ p['sa_w1'][i] + p['sa_b1'][i]) @ p['sa_w2'][i] + p['sa_b2'][i]
    return _ln(x + hdn, p['sa_n2g'][i], p['sa_n2b'][i])


def _cross_block(x, y, p, i):
    n, d = x.shape
    h = NUM_HEAD
    dh = d // h
    q = (x @ p['ca_wq'][i]).reshape(n, h, dh)
    k = (y @ p['ca_wk'][i]).reshape(-1, h, dh)
    v = (y @ p['ca_wv'][i]).reshape(-1, h, dh)
    att = jax.nn.softmax(jnp.einsum('nhd,mhd->hnm', q, k) / np.sqrt(dh), axis=-1)
    o = jnp.einsum('hnm,mhd->nhd', att, v).reshape(n, d) @ p['ca_wo'][i]
    x = _ln(x + o, p['ca_n1g'][i], p['ca_n1b'][i])
    hdn = jax.nn.relu(x @ p['ca_w1'][i] + p['ca_b1'][i]) @ p['ca_w2'][i] + p['ca_b2'][i]
    return _ln(x + hdn, p['ca_n2g'][i], p['ca_n2b'][i])


def kernel(ref_points, src_points, ref_feats, src_feats, Radis, params):
    nkd_r, nki_r = _knn(ref_points)
    nkd_s, nki_s = _knn(src_points)
    rge_r, cpe, _ = _geom_embed(ref_points, nkd_r, nki_r, params, True)
    rge_s, _ = _geom_embed(src_points, nkd_s, nki_s, params, False)
    rf = ref_feats @ params['in_w'] + params['in_b']
    sf = src_feats @ params['in_w'] + params['in_b']
    cpe_pool = cpe[:, 0].max(axis=1)
    oh_r = (nki_r[:, :, None] == jnp.arange(ref_points.shape[0])).astype(jnp.float32)
    oh_s = (nki_s[:, :, None] == jnp.arange(src_points.shape[0])).astype(jnp.float32)
    masks_list = []
    for i in range(BLOCKS):
        rf = _self_block(rf, rge_r, oh_r, params, i)
        sf = _self_block(sf, rge_s, oh_s, params, i)
        rf2 = _cross_block(rf + cpe_pool, sf, params, i)
        sf2 = _cross_block(sf, rf, params, i)
        rf, sf = rf2, sf2
        m = jax.nn.sigmoid((rf @ params['mask_w'][i]) @ sf.T / np.sqrt(HIDDEN))
        masks_list.append(m)
    rf = rf @ params['out_w'] + params['out_b']
    sf = sf @ params['out_w'] + params['out_b']
    return rf, sf, nki_r, nki_s, tuple(masks_list)


# speedup vs baseline: 1.3183x; 1.3183x over previous
import functools

import jax
import jax.numpy as jnp
import numpy as np
from jax import lax
from jax.experimental import pallas as pl
from jax.experimental.pallas import tpu as pltpu
from jax.experimental.pallas import tpu_sc as plsc

HIDDEN = 256
NUM_HEAD = 4
BLOCKS = 3
INPUT_DIM = 1024
OUTPUT_DIM = 256
BIN_SIZE_D = 0.2
BIN_SIZE_A = 15.0
ANGLE_K = 3
MAX_NEIGHBOOR = 16
GEODESIC_RADIS = 0.5
BIN_FACTOR_A = 180.0 / (BIN_SIZE_A * np.pi)
BIN_SIZE_GEO = BIN_SIZE_D / 2.0


def _knn_body(pts_ref, ptsT_ref, nkd_ref, nki_ref):
    p = pts_ref[...]
    pt = ptsT_ref[...]
    n = p.shape[0]
    d2 = jnp.zeros((n, n), jnp.float32)
    for d in range(3):
        diff = p[:, d : d + 1] - pt[d : d + 1, :]
        d2 = d2 + diff * diff
    cols = lax.broadcasted_iota(jnp.int32, (n, n), 1)
    dist = jnp.sqrt(jnp.maximum(d2, 1e-12))
    for k in range(MAX_NEIGHBOOR):
        m = jnp.min(dist, axis=1, keepdims=True)
        am = jnp.min(jnp.where(dist <= m, cols, n), axis=1, keepdims=True)
        nkd_ref[:, k : k + 1] = m
        nki_ref[:, k : k + 1] = am
        dist = jnp.where(cols == am, jnp.inf, dist)


def _knn(points, interpret=False):
    n = points.shape[0]
    return pl.pallas_call(
        _knn_body,
        out_shape=(
            jax.ShapeDtypeStruct((n, MAX_NEIGHBOOR), jnp.float32),
            jax.ShapeDtypeStruct((n, MAX_NEIGHBOOR), jnp.int32),
        ),
        interpret=interpret,
    )(points, points.T)


def _sin_embed(x, d=HIDDEN):
    div = jnp.exp(jnp.arange(0, d, 2, dtype=jnp.float32) * (-np.log(10000.0) / d))
    om = x[..., None] * div
    e = jnp.stack([jnp.sin(om), jnp.cos(om)], axis=-1)
    return e.reshape(x.shape + (d,))


_N = 1024
_K = MAX_NEIGHBOOR
_NW = 32
_CPW = _N // _NW


def _geo_sc_body(edgeT_hbm, idxT_hbm, out_hbm, edge_v, idx_v, a_v, b_v, chg_v,
                 sem):
    wid = lax.axis_index("s") * 2 + lax.axis_index("c")
    pltpu.sync_copy(edgeT_hbm, edge_v)
    pltpu.sync_copy(idxT_hbm, idx_v)
    lanes = jax.lax.iota(jnp.int32, 16)
    inf16 = jnp.full((16,), jnp.inf, jnp.float32)

    def init_g(g, _):
        rows = lanes + g * 16
        for c in range(_CPW):
            a_v[pl.ds(c * _N + g * 16, 16)] = jnp.where(
                rows == wid * _CPW + c, 0.0, inf16)
        return 0
    lax.fori_loop(0, _N // 16, init_g, 0)

    def relax(s_ref, d_ref, tracked):
        def g_body(g, chg):
            i0 = g * 16
            nv = [idx_v[k, pl.ds(i0, 16)] for k in range(_K)]
            ev = [edge_v[k, pl.ds(i0, 16)] for k in range(_K)]

            def c_body(cc, chg):
                base = cc * _N
                bvec = jnp.full((16,), base, jnp.int32)
                cand = [
                    ev[k] + plsc.load_gather(s_ref, [nv[k] + bvec])
                    for k in range(_K)
                ]
                while len(cand) > 1:
                    cand = [jnp.minimum(cand[2 * i], cand[2 * i + 1])
                            for i in range(len(cand) // 2)]
                old = s_ref[pl.ds(base + i0, 16)]
                acc = jnp.minimum(old, cand[0])
                d_ref[pl.ds(base + i0, 16)] = acc
                if tracked:
                    chg = jnp.where(acc != old, jnp.int32(1), chg)
                return chg

            return lax.fori_loop(0, _CPW, c_body, chg)

        return lax.fori_loop(0, _N // 16, g_body,
                             jnp.zeros((16,), jnp.int32))

    chg_v[...] = jnp.ones((16,), jnp.int32)

    def pair_body(t, conv):
        @pl.when(conv > 0)
        def _():
            relax(a_v, b_v, False)
            chg_v[...] = relax(b_v, a_v, True)

        return jnp.max(chg_v[...])

    lax.fori_loop(0, 16, pair_body, jnp.int32(1))
    pltpu.sync_copy(a_v, out_hbm.at[pl.ds(wid * _CPW * _N, _CPW * _N)])


def _geodesic_sc(edge_d, nki, interpret=False):
    mesh = plsc.VectorSubcoreMesh(core_axis_name="c", subcore_axis_name="s")
    kern = pl.kernel(
        _geo_sc_body,
        out_type=jax.ShapeDtypeStruct((_N * _N,), jnp.float32),
        mesh=mesh,
        compiler_params=pltpu.CompilerParams(
            needs_layout_passes=False, use_tc_tiling_on_sc=False),
        scratch_types=[
            pltpu.VMEM((_K, _N), jnp.float32),
            pltpu.VMEM((_K, _N), jnp.int32),
            pltpu.VMEM((_CPW * _N,), jnp.float32),
            pltpu.VMEM((_CPW * _N,), jnp.float32),
            pltpu.VMEM((16,), jnp.int32),
            pltpu.SemaphoreType.DMA,
        ],
        interpret=interpret,
    )
    return kern(edge_d.T, nki.T).reshape(_N, _N)


def _geodesic(nkd, nki, max_step=32):
    n = nkd.shape[0]
    edge_d = jnp.where(nkd <= GEODESIC_RADIS, nkd, jnp.inf)
    geoT = _geodesic_sc(edge_d, nki)
    return jnp.where(jnp.isinf(geoT), -1.0, geoT)


def _ln(x, g, b):
    mu = x.mean(-1, keepdims=True)
    v = ((x - mu) ** 2).mean(-1, keepdims=True)
    return (x - mu) / jnp.sqrt(v + 1e-5) * g + b


def _geom_embed(points, nkd, nki, oh, p, geodesic):
    masks = None
    hi = jax.lax.Precision.HIGHEST
    nkp = jnp.einsum('nkj,jc->nkc', oh, points, precision=hi)
    rde_idx = nkd / BIN_SIZE_D
    knn_pts = nkp[:, 1 : 1 + ANGLE_K, :]
    rv = (knn_pts - points[:, None, :]).reshape(-1, ANGLE_K * 3)
    anc_vec = nkp[:, None, :, :] - points[:, None, None, :]
    ref_vec = jnp.einsum('nkj,jc->nkc', oh, rv, precision=hi)
    ref_vec = ref_vec.reshape(-1, MAX_NEIGHBOOR, ANGLE_K, 3)[:, None]
    anc_vec = jnp.broadcast_to(anc_vec[:, :, :, None, :], ref_vec.shape)
    sinv = jnp.linalg.norm(jnp.cross(ref_vec, anc_vec), axis=-1)
    cosv = jnp.sum(ref_vec * anc_vec, axis=-1)
    rae_idx = jnp.arctan2(sinv, cosv) * BIN_FACTOR_A
    rde = _sin_embed(rde_idx[:, None]) @ p['rde_w'] + p['rde_b']
    rae = _sin_embed(rae_idx) @ p['rae_w'] + p['rae_b']
    rae = rae.max(axis=3)
    rge = rde + rae
    if geodesic:
        geoW = _geodesic(nkd, nki)
        mctx = geoW.max(axis=0)
        mval = mctx.max()
        mctx = jnp.where(mctx < 0, mval, mctx)
        gd0 = jnp.take_along_axis(geoW, nki.T, axis=0).T
        gd = jnp.where(gd0 < 0, mctx[:, None], gd0) / BIN_SIZE_GEO
        rdi = _sin_embed(gd[:, None]) @ p['rdi_w'] + p['rdi_b'] + rde
        return rge, rdi, masks
    return rge, masks


def _self_block(x, emb, knn_oh, p, i):
    n, d = x.shape
    h = NUM_HEAD
    dh = d // h
    q = x @ p['sa_wq'][i]
    k = x @ p['sa_wk'][i]
    v = x @ p['sa_wv'][i]
    pe = emb[:, 0] @ p['sa_wp'][i]
    kn = jnp.einsum('nkj,jd->nkd', knn_oh, k,
                    precision=jax.lax.Precision.HIGHEST) + pe
    vn = jnp.einsum('nkj,jd->nkd', knn_oh, v,
                    precision=jax.lax.Precision.HIGHEST)
    qh = q.reshape(n, h, dh)
    knh = kn.reshape(n, -1, h, dh)
    vnh = vn.reshape(n, -1, h, dh)
    att = jax.nn.softmax(jnp.einsum('nhd,nkhd->nhk', qh, knh) / np.sqrt(dh), axis=-1)
    o = jnp.einsum('nhk,nkhd->nhd', att, vnh).reshape(n, d) @ p['sa_wo'][i]
    x = _ln(x + o, p['sa_n1g'][i], p['sa_n1b'][i])
    hdn = jax.nn.relu(x @ p['sa_w1'][i] + p['sa_b1'][i]) @ p['sa_w2'][i] + p['sa_b2'][i]
    return _ln(x + hdn, p['sa_n2g'][i], p['sa_n2b'][i])


def _cross_block(x, y, p, i):
    n, d = x.shape
    h = NUM_HEAD
    dh = d // h
    q = (x @ p['ca_wq'][i]).reshape(n, h, dh)
    k = (y @ p['ca_wk'][i]).reshape(-1, h, dh)
    v = (y @ p['ca_wv'][i]).reshape(-1, h, dh)
    att = jax.nn.softmax(jnp.einsum('nhd,mhd->hnm', q, k) / np.sqrt(dh), axis=-1)
    o = jnp.einsum('hnm,mhd->nhd', att, v).reshape(n, d) @ p['ca_wo'][i]
    x = _ln(x + o, p['ca_n1g'][i], p['ca_n1b'][i])
    hdn = jax.nn.relu(x @ p['ca_w1'][i] + p['ca_b1'][i]) @ p['ca_w2'][i] + p['ca_b2'][i]
    return _ln(x + hdn, p['ca_n2g'][i], p['ca_n2b'][i])


def kernel(ref_points, src_points, ref_feats, src_feats, Radis, params):
    nkd_r, nki_r = _knn(ref_points)
    nkd_s, nki_s = _knn(src_points)
    oh_r = (nki_r[:, :, None] == jnp.arange(ref_points.shape[0])).astype(jnp.float32)
    oh_s = (nki_s[:, :, None] == jnp.arange(src_points.shape[0])).astype(jnp.float32)
    rge_r, cpe, _ = _geom_embed(ref_points, nkd_r, nki_r, oh_r, params, True)
    rge_s, _ = _geom_embed(src_points, nkd_s, nki_s, oh_s, params, False)
    rf = ref_feats @ params['in_w'] + params['in_b']
    sf = src_feats @ params['in_w'] + params['in_b']
    cpe_pool = cpe[:, 0].max(axis=1)
    masks_list = []
    for i in range(BLOCKS):
        rf = _self_block(rf, rge_r, oh_r, params, i)
        sf = _self_block(sf, rge_s, oh_s, params, i)
        rf2 = _cross_block(rf + cpe_pool, sf, params, i)
        sf2 = _cross_block(sf, rf, params, i)
        rf, sf = rf2, sf2
        m = jax.nn.sigmoid((rf @ params['mask_w'][i]) @ sf.T / np.sqrt(HIDDEN))
        masks_list.append(m)
    rf = rf @ params['out_w'] + params['out_b']
    sf = sf @ params['out_w'] + params['out_b']
    return rf, sf, nki_r, nki_s, tuple(masks_list)


# speedup vs baseline: 1.3853x; 1.0509x over previous
import functools

import jax
import jax.numpy as jnp
import numpy as np
from jax import lax
from jax.experimental import pallas as pl
from jax.experimental.pallas import tpu as pltpu
from jax.experimental.pallas import tpu_sc as plsc

HIDDEN = 256
NUM_HEAD = 4
BLOCKS = 3
INPUT_DIM = 1024
OUTPUT_DIM = 256
BIN_SIZE_D = 0.2
BIN_SIZE_A = 15.0
ANGLE_K = 3
MAX_NEIGHBOOR = 16
GEODESIC_RADIS = 0.5
BIN_FACTOR_A = 180.0 / (BIN_SIZE_A * np.pi)
BIN_SIZE_GEO = BIN_SIZE_D / 2.0


def _knn_body(pts_ref, ptsT_ref, nkd_ref, nki_ref):
    p = pts_ref[...]
    pt = ptsT_ref[...]
    n = p.shape[0]
    d2 = jnp.zeros((n, n), jnp.float32)
    for d in range(3):
        diff = p[:, d : d + 1] - pt[d : d + 1, :]
        d2 = d2 + diff * diff
    cols = lax.broadcasted_iota(jnp.int32, (n, n), 1)
    dist = jnp.sqrt(jnp.maximum(d2, 1e-12))
    for k in range(MAX_NEIGHBOOR):
        m = jnp.min(dist, axis=1, keepdims=True)
        am = jnp.min(jnp.where(dist <= m, cols, n), axis=1, keepdims=True)
        nkd_ref[:, k : k + 1] = m
        nki_ref[:, k : k + 1] = am
        dist = jnp.where(cols == am, jnp.inf, dist)


def _knn(points, interpret=False):
    n = points.shape[0]
    return pl.pallas_call(
        _knn_body,
        out_shape=(
            jax.ShapeDtypeStruct((n, MAX_NEIGHBOOR), jnp.float32),
            jax.ShapeDtypeStruct((n, MAX_NEIGHBOOR), jnp.int32),
        ),
        interpret=interpret,
    )(points, points.T)


def _sin_embed(x, d=HIDDEN):
    div = jnp.exp(jnp.arange(0, d, 2, dtype=jnp.float32) * (-np.log(10000.0) / d))
    om = x[..., None] * div
    e = jnp.stack([jnp.sin(om), jnp.cos(om)], axis=-1)
    return e.reshape(x.shape + (d,))


def _sin_embed_mm(x, w, b, d=HIDDEN):
    div = jnp.exp(jnp.arange(0, d, 2, dtype=jnp.float32) * (-np.log(10000.0) / d))
    om = x[..., None] * div
    return jnp.sin(om) @ w[0::2] + jnp.cos(om) @ w[1::2] + b


_N = 1024
_K = MAX_NEIGHBOOR
_NW = 32
_CPW = _N // _NW


def _geo_sc_body(edgeT_hbm, idxT_hbm, out_hbm, edge_v, idx_v, a_v, b_v, chg_v,
                 sem):
    wid = lax.axis_index("s") * 2 + lax.axis_index("c")
    pltpu.sync_copy(edgeT_hbm, edge_v)
    pltpu.sync_copy(idxT_hbm, idx_v)
    lanes = jax.lax.iota(jnp.int32, 16)
    inf16 = jnp.full((16,), jnp.inf, jnp.float32)

    def init_g(g, _):
        rows = lanes + g * 16
        for c in range(_CPW):
            a_v[pl.ds(c * _N + g * 16, 16)] = jnp.where(
                rows == wid * _CPW + c, 0.0, inf16)
        return 0
    lax.fori_loop(0, _N // 16, init_g, 0)

    def relax(s_ref, d_ref, tracked):
        def g_body(g, chg):
            i0 = g * 16
            nv = [idx_v[k, pl.ds(i0, 16)] for k in range(_K)]
            ev = [edge_v[k, pl.ds(i0, 16)] for k in range(_K)]

            def c_body(cc, chg):
                base = cc * _N
                bvec = jnp.full((16,), base, jnp.int32)
                cand = [
                    ev[k] + plsc.load_gather(s_ref, [nv[k] + bvec])
                    for k in range(_K)
                ]
                while len(cand) > 1:
                    cand = [jnp.minimum(cand[2 * i], cand[2 * i + 1])
                            for i in range(len(cand) // 2)]
                old = s_ref[pl.ds(base + i0, 16)]
                acc = jnp.minimum(old, cand[0])
                d_ref[pl.ds(base + i0, 16)] = acc
                if tracked:
                    chg = jnp.where(acc != old, jnp.int32(1), chg)
                return chg

            return lax.fori_loop(0, _CPW, c_body, chg)

        return lax.fori_loop(0, _N // 16, g_body,
                             jnp.zeros((16,), jnp.int32))

    chg_v[...] = jnp.ones((16,), jnp.int32)

    def pair_body(t, conv):
        @pl.when(conv > 0)
        def _():
            relax(a_v, b_v, False)
            chg_v[...] = relax(b_v, a_v, True)

        return jnp.max(chg_v[...])

    lax.fori_loop(0, 16, pair_body, jnp.int32(1))
    pltpu.sync_copy(a_v, out_hbm.at[pl.ds(wid * _CPW * _N, _CPW * _N)])


def _geodesic_sc(edge_d, nki, interpret=False):
    mesh = plsc.VectorSubcoreMesh(core_axis_name="c", subcore_axis_name="s")
    kern = pl.kernel(
        _geo_sc_body,
        out_type=jax.ShapeDtypeStruct((_N * _N,), jnp.float32),
        mesh=mesh,
        compiler_params=pltpu.CompilerParams(
            needs_layout_passes=False, use_tc_tiling_on_sc=False),
        scratch_types=[
            pltpu.VMEM((_K, _N), jnp.float32),
            pltpu.VMEM((_K, _N), jnp.int32),
            pltpu.VMEM((_CPW * _N,), jnp.float32),
            pltpu.VMEM((_CPW * _N,), jnp.float32),
            pltpu.VMEM((16,), jnp.int32),
            pltpu.SemaphoreType.DMA,
        ],
        interpret=interpret,
    )
    return kern(edge_d.T, nki.T).reshape(_N, _N)


def _geodesic(nkd, nki, max_step=32):
    n = nkd.shape[0]
    edge_d = jnp.where(nkd <= GEODESIC_RADIS, nkd, jnp.inf)
    geoT = _geodesic_sc(edge_d, nki)
    return jnp.where(jnp.isinf(geoT), -1.0, geoT)


def _ln(x, g, b):
    mu = x.mean(-1, keepdims=True)
    v = ((x - mu) ** 2).mean(-1, keepdims=True)
    return (x - mu) / jnp.sqrt(v + 1e-5) * g + b


def _geom_embed(points, nkd, nki, oh, p, geodesic):
    masks = None
    hi = jax.lax.Precision.HIGHEST
    nkp = jnp.einsum('nkj,jc->nkc', oh, points, precision=hi)
    rde_idx = nkd / BIN_SIZE_D
    knn_pts = nkp[:, 1 : 1 + ANGLE_K, :]
    rv = (knn_pts - points[:, None, :]).reshape(-1, ANGLE_K * 3)
    anc_vec = nkp[:, None, :, :] - points[:, None, None, :]
    ref_vec = jnp.einsum('nkj,jc->nkc', oh, rv, precision=hi)
    ref_vec = ref_vec.reshape(-1, MAX_NEIGHBOOR, ANGLE_K, 3)[:, None]
    anc_vec = jnp.broadcast_to(anc_vec[:, :, :, None, :], ref_vec.shape)
    sinv = jnp.linalg.norm(jnp.cross(ref_vec, anc_vec), axis=-1)
    cosv = jnp.sum(ref_vec * anc_vec, axis=-1)
    rae_idx = jnp.arctan2(sinv, cosv) * BIN_FACTOR_A
    rde = _sin_embed_mm(rde_idx[:, None], p['rde_w'], p['rde_b'])
    rae = _sin_embed_mm(rae_idx, p['rae_w'], p['rae_b'])
    rae = rae.max(axis=3)
    rge = rde + rae
    if geodesic:
        geoW = _geodesic(nkd, nki)
        mctx = geoW.max(axis=0)
        mval = mctx.max()
        mctx = jnp.where(mctx < 0, mval, mctx)
        gd0 = jnp.take_along_axis(geoW, nki.T, axis=0).T
        gd = jnp.where(gd0 < 0, mctx[:, None], gd0) / BIN_SIZE_GEO
        rdi = _sin_embed_mm(gd[:, None], p['rdi_w'], p['rdi_b']) + rde
        return rge, rdi, masks
    return rge, masks


def _self_block(x, emb, knn_oh, p, i):
    n, d = x.shape
    h = NUM_HEAD
    dh = d // h
    q = x @ p['sa_wq'][i]
    k = x @ p['sa_wk'][i]
    v = x @ p['sa_wv'][i]
    pe = emb[:, 0] @ p['sa_wp'][i]
    kn = jnp.einsum('nkj,jd->nkd', knn_oh, k,
                    precision=jax.lax.Precision.HIGHEST) + pe
    vn = jnp.einsum('nkj,jd->nkd', knn_oh, v,
                    precision=jax.lax.Precision.HIGHEST)
    qh = q.reshape(n, h, dh)
    knh = kn.reshape(n, -1, h, dh)
    vnh = vn.reshape(n, -1, h, dh)
    att = jax.nn.softmax(jnp.einsum('nhd,nkhd->nhk', qh, knh) / np.sqrt(dh), axis=-1)
    o = jnp.einsum('nhk,nkhd->nhd', att, vnh).reshape(n, d) @ p['sa_wo'][i]
    x = _ln(x + o, p['sa_n1g'][i], p['sa_n1b'][i])
    hdn = jax.nn.relu(x @ p['sa_w1'][i] + p['sa_b1'][i]) @ p['sa_w2'][i] + p['sa_b2'][i]
    return _ln(x + hdn, p['sa_n2g'][i], p['sa_n2b'][i])


def _cross_block(x, y, p, i):
    n, d = x.shape
    h = NUM_HEAD
    dh = d // h
    q = (x @ p['ca_wq'][i]).reshape(n, h, dh)
    k = (y @ p['ca_wk'][i]).reshape(-1, h, dh)
    v = (y @ p['ca_wv'][i]).reshape(-1, h, dh)
    att = jax.nn.softmax(jnp.einsum('nhd,mhd->hnm', q, k) / np.sqrt(dh), axis=-1)
    o = jnp.einsum('hnm,mhd->nhd', att, v).reshape(n, d) @ p['ca_wo'][i]
    x = _ln(x + o, p['ca_n1g'][i], p['ca_n1b'][i])
    hdn = jax.nn.relu(x @ p['ca_w1'][i] + p['ca_b1'][i]) @ p['ca_w2'][i] + p['ca_b2'][i]
    return _ln(x + hdn, p['ca_n2g'][i], p['ca_n2b'][i])


def kernel(ref_points, src_points, ref_feats, src_feats, Radis, params):
    nkd_r, nki_r = _knn(ref_points)
    nkd_s, nki_s = _knn(src_points)
    oh_r = (nki_r[:, :, None] == jnp.arange(ref_points.shape[0])).astype(jnp.float32)
    oh_s = (nki_s[:, :, None] == jnp.arange(src_points.shape[0])).astype(jnp.float32)
    rge_r, cpe, _ = _geom_embed(ref_points, nkd_r, nki_r, oh_r, params, True)
    rge_s, _ = _geom_embed(src_points, nkd_s, nki_s, oh_s, params, False)
    rf = ref_feats @ params['in_w'] + params['in_b']
    sf = src_feats @ params['in_w'] + params['in_b']
    cpe_pool = cpe[:, 0].max(axis=1)
    masks_list = []
    for i in range(BLOCKS):
        rf = _self_block(rf, rge_r, oh_r, params, i)
        sf = _self_block(sf, rge_s, oh_s, params, i)
        rf2 = _cross_block(rf + cpe_pool, sf, params, i)
        sf2 = _cross_block(sf, rf, params, i)
        rf, sf = rf2, sf2
        m = jax.nn.sigmoid((rf @ params['mask_w'][i]) @ sf.T / np.sqrt(HIDDEN))
        masks_list.append(m)
    rf = rf @ params['out_w'] + params['out_b']
    sf = sf @ params['out_w'] + params['out_b']
    return rf, sf, nki_r, nki_s, tuple(masks_list)
